# baseline (speedup 1.0000x reference)
import functools

import jax
import jax.numpy as jnp
from jax.experimental import pallas as pl
from jax.experimental.pallas import tpu as pltpu


def _ffn_kernel(x_ref, g_ref, bt_ref, w1_ref, b1_ref, w2_ref, b2_ref, o_ref,
                w1b_ref, w2b_ref, p_ref, st1_ref, st2_ref, xb_ref, yb_ref,
                sem1_ref, sem2_ref, semp_ref, semx_ref, semy_ref,
                *, eps, c1, c2, row_tile, n_tiles):
    E = w1_ref.shape[0]
    M = w2_ref.shape[0]
    n1 = E // c1
    n2 = M // c2

    def xin_start(t, buf):
        pltpu.make_async_copy(x_ref.at[pl.ds(t * row_tile, row_tile), :],
                              xb_ref.at[buf], semx_ref.at[buf]).start()

    def xin_wait(buf):
        pltpu.make_async_copy(xb_ref.at[buf], xb_ref.at[buf],
                              semx_ref.at[buf]).wait()

    def yout_start(t, buf):
        pltpu.make_async_copy(yb_ref.at[buf],
                              o_ref.at[pl.ds(t * row_tile, row_tile), :],
                              semy_ref.at[buf]).start()

    def yout_wait(buf):
        pltpu.make_async_copy(yb_ref.at[buf], yb_ref.at[buf],
                              semy_ref.at[buf]).wait()

    xin_start(0, 0)
    if n_tiles > 1:
        xin_start(1, 1)

    pltpu.make_async_copy(g_ref, p_ref.at[0, :, :E], semp_ref.at[0]).start()
    pltpu.make_async_copy(bt_ref, p_ref.at[1, :, :E], semp_ref.at[1]).start()
    pltpu.make_async_copy(b2_ref, p_ref.at[2, :, :E], semp_ref.at[2]).start()
    pltpu.make_async_copy(b1_ref, p_ref.at[3, :, :M], semp_ref.at[3]).start()

    def start1(c, buf):
        pltpu.make_async_copy(w1_ref.at[pl.ds(c * c1, c1), :],
                              st1_ref.at[buf], sem1_ref.at[buf]).start()

    def start2(c, buf):
        pltpu.make_async_copy(w2_ref.at[pl.ds(c * c2, c2), :],
                              st2_ref.at[buf], sem2_ref.at[buf]).start()

    start1(0, 0)
    if n1 > 1:
        start1(1, 1)
    n2_started = 0
    for c in range(n1):
        buf = c % 2
        pltpu.make_async_copy(st1_ref.at[buf], st1_ref.at[buf],
                              sem1_ref.at[buf]).wait()
        if c + 2 < n1:
            start1(c + 2, buf)
        elif n2_started < min(2, n2):
            start2(n2_started, n2_started)
            n2_started += 1
        w1b_ref[pl.ds(c * c1, c1), :] = st1_ref[buf].astype(jnp.bfloat16)
    for c in range(n2_started, min(2, n2)):
        start2(c, c)
    for c in range(n2):
        buf = c % 2
        pltpu.make_async_copy(st2_ref.at[buf], st2_ref.at[buf],
                              sem2_ref.at[buf]).wait()
        if c + 2 < n2:
            start2(c + 2, buf)
        w2b_ref[pl.ds(c * c2, c2), :] = st2_ref[buf].astype(jnp.bfloat16)

    pltpu.make_async_copy(g_ref, p_ref.at[0, :, :E], semp_ref.at[0]).wait()
    pltpu.make_async_copy(bt_ref, p_ref.at[1, :, :E], semp_ref.at[1]).wait()
    pltpu.make_async_copy(b2_ref, p_ref.at[2, :, :E], semp_ref.at[2]).wait()
    pltpu.make_async_copy(b1_ref, p_ref.at[3, :, :M], semp_ref.at[3]).wait()

    gamma = p_ref[0, :, :E]
    beta = p_ref[1, :, :E]
    b2v = p_ref[2, :, :E]
    b1v = p_ref[3, :, :M]

    def body(t, _):
        cur = jax.lax.rem(t, 2)

        xin_wait(cur)
        x = xb_ref[cur].astype(jnp.float32)
        mean = jnp.mean(x, axis=-1, keepdims=True)
        xc = x - mean
        var = jnp.mean(xc * xc, axis=-1, keepdims=True)
        xn = xc * jax.lax.rsqrt(var + eps)
        xn = xn * gamma + beta

        h = jnp.dot(xn.astype(jnp.bfloat16), w1b_ref[...],
                    preferred_element_type=jnp.float32)
        h = jax.nn.gelu(h + b1v, approximate=True)

        y = jnp.dot(h.astype(jnp.bfloat16), w2b_ref[...],
                    preferred_element_type=jnp.float32)

        @pl.when(t >= 2)
        def _():
            yout_wait(cur)
        yb_ref[cur] = (y + b2v).astype(yb_ref.dtype)
        yout_start(t, cur)

        @pl.when(t + 2 < n_tiles)
        def _():
            xin_start(t + 2, cur)
        return ()

    jax.lax.fori_loop(0, n_tiles, body, ())
    yout_wait(jax.lax.rem(n_tiles - 2, 2))
    yout_wait(jax.lax.rem(n_tiles - 1, 2))


def kernel(x, gamma, beta, w1, b1, w2, b2, *, eps=1e-5, row_tile=512,
           c1=128, c2=512, interpret=False):
    B, S, E = x.shape
    M = w1.shape[1]
    R = B * S

    x2 = x.reshape(R, E)
    R_pad = ((max(R, 1) + row_tile - 1) // row_tile) * row_tile
    if R_pad != R:
        x2 = jnp.pad(x2, ((0, R_pad - R), (0, 0)))
    T = R_pad // row_tile

    g2 = gamma.reshape(1, E).astype(jnp.float32)
    bt2 = beta.reshape(1, E).astype(jnp.float32)
    b1_2 = b1.reshape(1, M).astype(jnp.float32)
    b2_2 = b2.reshape(1, E).astype(jnp.float32)

    out = pl.pallas_call(
        functools.partial(_ffn_kernel, eps=eps, c1=c1, c2=c2,
                          row_tile=row_tile, n_tiles=T),
        out_shape=jax.ShapeDtypeStruct((R_pad, E), x.dtype),
        in_specs=[pl.BlockSpec(memory_space=pl.ANY)] * 7,
        out_specs=pl.BlockSpec(memory_space=pl.ANY),
        scratch_shapes=[
            pltpu.VMEM((E, M), jnp.bfloat16),
            pltpu.VMEM((M, E), jnp.bfloat16),
            pltpu.VMEM((4, 1, max(M, E)), jnp.float32),
            pltpu.VMEM((2, c1, M), jnp.float32),
            pltpu.VMEM((2, c2, E), jnp.float32),
            pltpu.VMEM((2, row_tile, E), jnp.float32),
            pltpu.VMEM((2, row_tile, E), jnp.float32),
            pltpu.SemaphoreType.DMA((2,)),
            pltpu.SemaphoreType.DMA((2,)),
            pltpu.SemaphoreType.DMA((4,)),
            pltpu.SemaphoreType.DMA((2,)),
            pltpu.SemaphoreType.DMA((2,)),
        ],
        compiler_params=pltpu.CompilerParams(
            vmem_limit_bytes=56 * 1024 * 1024,
        ),
        cost_estimate=pl.CostEstimate(
            flops=int(4 * R * E * M),
            transcendentals=int(R * M),
            bytes_accessed=int(R * E * 4 + R * E * 4 + 2 * E * M * 4),
        ),
        interpret=interpret,
    )(x2, g2, bt2, w1, b1_2, w2, b2_2)

    return out[:R].reshape(B, S, E)


# speedup vs baseline: 1.1610x; 1.1610x over previous
import functools

import jax
import jax.numpy as jnp
from jax.experimental import pallas as pl
from jax.experimental.pallas import tpu as pltpu


def _ffn_kernel(x_ref, g_ref, bt_ref, w1_ref, b1_ref, w2_ref, b2_ref, o_ref,
                w1b_ref, w2b_ref, p_ref, st1_ref, st2_ref,
                sem1_ref, sem2_ref, semp_ref, *, eps, c1, c2):
    E = w1_ref.shape[0]
    M = w2_ref.shape[0]
    n1 = E // c1
    n2 = M // c2

    @pl.when(pl.program_id(0) == 0)
    def _load_weights():
        pltpu.make_async_copy(g_ref, p_ref.at[0, :, :E], semp_ref.at[0]).start()
        pltpu.make_async_copy(bt_ref, p_ref.at[1, :, :E], semp_ref.at[1]).start()
        pltpu.make_async_copy(b2_ref, p_ref.at[2, :, :E], semp_ref.at[2]).start()
        pltpu.make_async_copy(b1_ref, p_ref.at[3, :, :M], semp_ref.at[3]).start()

        def start1(c, buf):
            pltpu.make_async_copy(w1_ref.at[pl.ds(c * c1, c1), :],
                                  st1_ref.at[buf], sem1_ref.at[buf]).start()

        def start2(c, buf):
            pltpu.make_async_copy(w2_ref.at[pl.ds(c * c2, c2), :],
                                  st2_ref.at[buf], sem2_ref.at[buf]).start()

        start1(0, 0)
        if n1 > 1:
            start1(1, 1)
        n2_started = 0
        for c in range(n1):
            buf = c % 2
            pltpu.make_async_copy(st1_ref.at[buf], st1_ref.at[buf],
                                  sem1_ref.at[buf]).wait()
            if c + 2 < n1:
                start1(c + 2, buf)
            elif n2_started < min(2, n2):
                start2(n2_started, n2_started)
                n2_started += 1
            w1b_ref[pl.ds(c * c1, c1), :] = st1_ref[buf].astype(jnp.bfloat16)
        for c in range(n2_started, min(2, n2)):
            start2(c, c)
        for c in range(n2):
            buf = c % 2
            pltpu.make_async_copy(st2_ref.at[buf], st2_ref.at[buf],
                                  sem2_ref.at[buf]).wait()
            if c + 2 < n2:
                start2(c + 2, buf)
            w2b_ref[pl.ds(c * c2, c2), :] = st2_ref[buf].astype(jnp.bfloat16)

        pltpu.make_async_copy(g_ref, p_ref.at[0, :, :E], semp_ref.at[0]).wait()
        pltpu.make_async_copy(bt_ref, p_ref.at[1, :, :E], semp_ref.at[1]).wait()
        pltpu.make_async_copy(b2_ref, p_ref.at[2, :, :E], semp_ref.at[2]).wait()
        pltpu.make_async_copy(b1_ref, p_ref.at[3, :, :M], semp_ref.at[3]).wait()

    gamma = p_ref[0, :, :E]
    beta = p_ref[1, :, :E]
    b2v = p_ref[2, :, :E]
    b1v = p_ref[3, :, :M]

    x = x_ref[...].astype(jnp.float32)
    mean = jnp.mean(x, axis=-1, keepdims=True)
    xc = x - mean
    var = jnp.mean(xc * xc, axis=-1, keepdims=True)
    xn = xc * jax.lax.rsqrt(var + eps)
    xn = xn * gamma + beta

    h = jnp.dot(xn.astype(jnp.bfloat16), w1b_ref[...],
                preferred_element_type=jnp.float32)
    h = jax.nn.gelu(h + b1v, approximate=True)

    y = jnp.dot(h.astype(jnp.bfloat16), w2b_ref[...],
                preferred_element_type=jnp.float32)
    o_ref[...] = (y + b2v).astype(o_ref.dtype)


def kernel(x, gamma, beta, w1, b1, w2, b2, *, eps=1e-5, row_tile=512,
           c1=128, c2=512):
    B, S, E = x.shape
    M = w1.shape[1]
    R = B * S

    x2 = x.reshape(R, E)
    R_pad = ((max(R, 1) + row_tile - 1) // row_tile) * row_tile
    if R_pad != R:
        x2 = jnp.pad(x2, ((0, R_pad - R), (0, 0)))
    T = R_pad // row_tile

    g2 = gamma.reshape(1, E).astype(jnp.float32)
    bt2 = beta.reshape(1, E).astype(jnp.float32)
    b1_2 = b1.reshape(1, M).astype(jnp.float32)
    b2_2 = b2.reshape(1, E).astype(jnp.float32)

    out = pl.pallas_call(
        functools.partial(_ffn_kernel, eps=eps, c1=c1, c2=c2),
        out_shape=jax.ShapeDtypeStruct((R_pad, E), x.dtype),
        grid=(T,),
        in_specs=[
            pl.BlockSpec((row_tile, E), lambda i: (i, 0)),
            pl.BlockSpec(memory_space=pl.ANY),
            pl.BlockSpec(memory_space=pl.ANY),
            pl.BlockSpec(memory_space=pl.ANY),
            pl.BlockSpec(memory_space=pl.ANY),
            pl.BlockSpec(memory_space=pl.ANY),
            pl.BlockSpec(memory_space=pl.ANY),
        ],
        out_specs=pl.BlockSpec((row_tile, E), lambda i: (i, 0)),
        scratch_shapes=[
            pltpu.VMEM((E, M), jnp.bfloat16),
            pltpu.VMEM((M, E), jnp.bfloat16),
            pltpu.VMEM((4, 1, max(M, E)), jnp.float32),
            pltpu.VMEM((2, c1, M), jnp.float32),
            pltpu.VMEM((2, c2, E), jnp.float32),
            pltpu.SemaphoreType.DMA((2,)),
            pltpu.SemaphoreType.DMA((2,)),
            pltpu.SemaphoreType.DMA((4,)),
        ],
        compiler_params=pltpu.CompilerParams(
            dimension_semantics=("arbitrary",),
            vmem_limit_bytes=56 * 1024 * 1024,
        ),
        cost_estimate=pl.CostEstimate(
            flops=int(4 * R * E * M),
            transcendentals=int(R * M),
            bytes_accessed=int(R * E * 4 + R * E * 4 + 2 * E * M * 4),
        ),
    )(x2, g2, bt2, w1, b1_2, w2, b2_2)

    return out[:R].reshape(B, S, E)
